# Initial kernel scaffold; baseline (speedup 1.0000x reference)
import jax
import jax.numpy as jnp
from jax.experimental import pallas as pl


def kernel(p, x, o, Wq, bq, Wk, bk, Wv, bv, Wp1, bp1, gP, betaP, Wp2, bp2, g1, beta1, W1, b1, g2, beta2, W2, b2):
    raise NotImplementedError("write your pallas kernel here")


# speedup vs baseline: 4.2318x; 4.2318x over previous
import functools

import jax
import jax.numpy as jnp
from jax import lax
from jax.experimental import pallas as pl
from jax.experimental.pallas import tpu as pltpu
from jax.experimental.pallas import tpu_sc as plsc

N = 10000
NP = 10240
NS = 16
D = 128
WD = 16
EPS = 1e-5
M = float(N * NS)
PADC = 1e4
BIG = 3e38


def _qkv_body(x_ref, w_ref, b_ref, o_ref):
    o_ref[...] = (
        jnp.dot(x_ref[...], w_ref[...], preferred_element_type=jnp.float32)
        + b_ref[...]
    )


def _qkv(xp, wqkv, bqkv):
    R = 512
    return pl.pallas_call(
        _qkv_body,
        grid=(NP // R,),
        in_specs=[
            pl.BlockSpec((R, D), lambda i: (i, 0)),
            pl.BlockSpec((D, 3 * D), lambda i: (0, 0)),
            pl.BlockSpec((1, 3 * D), lambda i: (0, 0)),
        ],
        out_specs=pl.BlockSpec((R, 3 * D), lambda i: (i, 0)),
        out_shape=jax.ShapeDtypeStruct((NP, 3 * D), jnp.float32),
    )(xp, wqkv, bqkv)


_KR = 256


def _knn_body(pb_ref, pt_ref, o_ref, scr):
    pb = pb_ref[...]
    pt = pt_ref[...]
    sqr = jnp.sum(pb * pb, axis=1, keepdims=True)
    sqc = jnp.sum(pt * pt, axis=0, keepdims=True)
    mm = lax.dot_general(pb, pt, (((1,), (0,)), ((), ())),
                         preferred_element_type=jnp.float32)
    scr[...] = sqr + sqc - 2.0 * mm
    cols = lax.broadcasted_iota(jnp.int32, (_KR, NP), 1).astype(jnp.float32)
    outs = []
    for _ in range(NS):
        d = scr[...]
        m = jnp.min(d, axis=1, keepdims=True)
        j = jnp.min(jnp.where(d == m, cols, BIG), axis=1, keepdims=True)
        outs.append(j)
        scr[...] = jnp.where(cols == j, BIG, d)
    o_ref[...] = jnp.concatenate(outs, axis=1).astype(jnp.int32)


def _knn(pp, ppt):
    return pl.pallas_call(
        _knn_body,
        grid=(NP // _KR,),
        in_specs=[
            pl.BlockSpec((_KR, 3), lambda i: (i, 0)),
            pl.BlockSpec((3, NP), lambda i: (0, 0)),
        ],
        out_specs=pl.BlockSpec((_KR, NS), lambda i: (i, 0)),
        out_shape=jax.ShapeDtypeStruct((NP, NS), jnp.int32),
        scratch_shapes=[pltpu.VMEM((_KR, NP), jnp.float32)],
    )(pp, ppt)


def _sc_counts(idxp, ones_h, zeros_h):
    total = idxp.shape[0]
    tpc = 16
    bpt = total // tpc
    ch = 128
    nch = bpt // ch
    mesh = plsc.VectorSubcoreMesh(core_axis_name="c", subcore_axis_name="s")

    @functools.partial(
        pl.kernel,
        mesh=mesh,
        out_type=jax.ShapeDtypeStruct((NP,), jnp.float32),
        scratch_types=[
            pltpu.VMEM((ch,), jnp.int32),
            pltpu.VMEM((ch,), jnp.float32),
            pltpu.VMEM_SHARED((NP,), jnp.float32),
        ],
    )
    def k(idx_hbm, ones_hbm, zeros_hbm, out_hbm, idx_v, ones_v, shared):
        cid = lax.axis_index("c")
        sid = lax.axis_index("s")

        @pl.when(cid == 0)
        def _():
            @pl.when(sid == 0)
            def _():
                pltpu.sync_copy(zeros_hbm, shared)

            plsc.subcore_barrier()
            pltpu.sync_copy(ones_hbm, ones_v)
            base = sid * bpt

            def chunk(c, carry):
                off = base + c * ch
                pltpu.sync_copy(idx_hbm.at[pl.ds(off, ch)], idx_v)
                pltpu.sync_copy(ones_v, shared.at[idx_v], add=True)
                return carry

            lax.fori_loop(0, nch, chunk, 0)
            plsc.subcore_barrier()

            @pl.when(sid == 0)
            def _():
                pltpu.sync_copy(shared, out_hbm)

    return k(idxp, ones_h, zeros_h)


def _sc_gather(ab, idxp):
    total = idxp.shape[0]
    nw = 32
    bpw = total // nw
    ch = 128
    nch = bpw // ch
    mesh = plsc.VectorSubcoreMesh(core_axis_name="c", subcore_axis_name="s")

    @functools.partial(
        pl.kernel,
        mesh=mesh,
        out_type=jax.ShapeDtypeStruct((total, 2 * D), jnp.float32),
        scratch_types=[
            pltpu.VMEM((ch,), jnp.int32),
            pltpu.VMEM((ch, 2 * D), jnp.float32),
            pltpu.SemaphoreType.DMA,
        ],
    )
    def k(ab_hbm, idx_hbm, out_hbm, idx_v, rows_v, sem):
        wid = lax.axis_index("s") * 2 + lax.axis_index("c")
        base = wid * bpw

        def chunk(c, carry):
            off = base + c * ch
            pltpu.sync_copy(idx_hbm.at[pl.ds(off, ch)], idx_v)
            pltpu.async_copy(ab_hbm.at[idx_v], rows_v, sem).wait()
            pltpu.sync_copy(rows_v, out_hbm.at[pl.ds(off, ch)])
            return carry

        lax.fori_loop(0, nch, chunk, 0)

    return k(ab, idxp)


_SR = 1024


def _statsp_body(p_ref, c_ref, wp1_ref, bp1_ref, o_ref):
    i = pl.program_id(0)
    pb = p_ref[...]
    c = c_ref[...]
    rows = (lax.broadcasted_iota(jnp.int32, (_SR, 1), 0).astype(jnp.float32)
            + i * float(_SR))
    cm = jnp.where(rows < float(N), c, 0.0)
    h = lax.dot_general(pb, wp1_ref[...], (((1,), (0,)), ((), ())),
                        preferred_element_type=jnp.float32) + bp1_ref[...]
    s0 = jnp.sum(cm * h, axis=0, keepdims=True)
    s1 = jnp.sum(cm * h * h, axis=0, keepdims=True)
    z = jnp.zeros((1, D - 3), jnp.float32)
    part = jnp.concatenate(
        [jnp.concatenate([s0, z], axis=1),
         jnp.concatenate([s1, z], axis=1),
         jnp.zeros((6, D), jnp.float32)],
        axis=0,
    )

    @pl.when(i == 0)
    def _():
        o_ref[...] = jnp.zeros_like(o_ref)

    o_ref[...] += part


def _statsp(pp, counts_col, wp1, bp1):
    return pl.pallas_call(
        _statsp_body,
        grid=(NP // _SR,),
        in_specs=[
            pl.BlockSpec((_SR, 3), lambda i: (i, 0)),
            pl.BlockSpec((_SR, 1), lambda i: (i, 0)),
            pl.BlockSpec((3, 3), lambda i: (0, 0)),
            pl.BlockSpec((1, 3), lambda i: (0, 0)),
        ],
        out_specs=pl.BlockSpec((8, D), lambda i: (0, 0)),
        out_shape=jax.ShapeDtypeStruct((8, D), jnp.float32),
    )(pp, counts_col, wp1, bp1)


_AR = 512


def _a2_body(p_ref, k_ref, v_ref, sp_ref, gp_ref, bP_ref, wp1_ref, bp1_ref,
             wp2_ref, bp2_ref, o_ref):
    s = sp_ref[...]
    mu = s[0:1, 0:3] / M
    var = s[1:2, 0:3] / M - mu * mu
    alpha = gp_ref[...] * lax.rsqrt(var + EPS)
    shift = bP_ref[...] - mu * alpha
    pb = p_ref[...]
    h = lax.dot_general(pb, wp1_ref[...], (((1,), (0,)), ((), ())),
                        preferred_element_type=jnp.float32) + bp1_ref[...]
    r = jnp.maximum(h * alpha + shift, 0.0)
    pr = lax.dot_general(r, wp2_ref[...], (((1,), (0,)), ((), ())),
                         preferred_element_type=jnp.float32) + bp2_ref[...]
    o_ref[...] = jnp.concatenate([k_ref[...] + pr, v_ref[...] + pr], axis=1)


def _a2(pp, qkv, statsp, gp, betap, wp1, bp1, wp2, bp2):
    return pl.pallas_call(
        _a2_body,
        grid=(NP // _AR,),
        in_specs=[
            pl.BlockSpec((_AR, 3), lambda i: (i, 0)),
            pl.BlockSpec((_AR, D), lambda i: (i, 1)),
            pl.BlockSpec((_AR, D), lambda i: (i, 2)),
            pl.BlockSpec((8, D), lambda i: (0, 0)),
            pl.BlockSpec((1, 3), lambda i: (0, 0)),
            pl.BlockSpec((1, 3), lambda i: (0, 0)),
            pl.BlockSpec((3, 3), lambda i: (0, 0)),
            pl.BlockSpec((1, 3), lambda i: (0, 0)),
            pl.BlockSpec((3, D), lambda i: (0, 0)),
            pl.BlockSpec((1, D), lambda i: (0, 0)),
        ],
        out_specs=pl.BlockSpec((_AR, 2 * D), lambda i: (i, 0)),
        out_shape=jax.ShapeDtypeStruct((NP, 2 * D), jnp.float32),
    )(pp, qkv, qkv, statsp, gp, betap, wp1, bp1, wp2, bp2)


_BR = 200


def _statsb_body(g_ref, q_ref, o_ref):
    w = g_ref[...] - q_ref[...][:, None, :]
    s0 = jnp.sum(w, axis=(0, 1))[None, :]
    s1 = jnp.sum(w * w, axis=(0, 1))[None, :]
    part = jnp.concatenate([s0, s1, jnp.zeros((6, D), jnp.float32)], axis=0)

    @pl.when(pl.program_id(0) == 0)
    def _():
        o_ref[...] = jnp.zeros_like(o_ref)

    o_ref[...] += part


def _statsb(g3, qkv):
    return pl.pallas_call(
        _statsb_body,
        grid=(N // _BR,),
        in_specs=[
            pl.BlockSpec((_BR, NS, D), lambda i: (i, 0, 0)),
            pl.BlockSpec((_BR, D), lambda i: (i, 0)),
        ],
        out_specs=pl.BlockSpec((8, D), lambda i: (0, 0)),
        out_shape=jax.ShapeDtypeStruct((8, D), jnp.float32),
    )(g3, qkv)


def _passc_body(g_ref, q_ref, st_ref, g1_ref, be1_ref, w1_ref, b1_ref,
                o_ref, so_ref):
    st = st_ref[...]
    mu = st[0:1, :] / M
    var = st[1:2, :] / M - mu * mu
    alpha = g1_ref[...] * lax.rsqrt(var + EPS)
    shift = be1_ref[...] - mu * alpha
    w = g_ref[...] - q_ref[...][:, None, :]
    w = jnp.maximum(w * alpha[None] + shift[None], 0.0)
    w2 = lax.dot_general(w.reshape(_BR * NS, D), w1_ref[...],
                         (((1,), (0,)), ((), ())),
                         preferred_element_type=jnp.float32) + b1_ref[...]
    o_ref[...] = w2
    s0 = jnp.sum(w2, axis=0, keepdims=True)
    s1 = jnp.sum(w2 * w2, axis=0, keepdims=True)
    part = jnp.concatenate([s0, s1, jnp.zeros((6, WD), jnp.float32)], axis=0)

    @pl.when(pl.program_id(0) == 0)
    def _():
        so_ref[...] = jnp.zeros_like(so_ref)

    so_ref[...] += part


def _passc(g3, qkv, stats1, g1, beta1, w1, b1):
    return pl.pallas_call(
        _passc_body,
        grid=(N // _BR,),
        in_specs=[
            pl.BlockSpec((_BR, NS, D), lambda i: (i, 0, 0)),
            pl.BlockSpec((_BR, D), lambda i: (i, 0)),
            pl.BlockSpec((8, D), lambda i: (0, 0)),
            pl.BlockSpec((1, D), lambda i: (0, 0)),
            pl.BlockSpec((1, D), lambda i: (0, 0)),
            pl.BlockSpec((D, WD), lambda i: (0, 0)),
            pl.BlockSpec((1, WD), lambda i: (0, 0)),
        ],
        out_specs=[
            pl.BlockSpec((_BR * NS, WD), lambda i: (i, 0)),
            pl.BlockSpec((8, WD), lambda i: (0, 0)),
        ],
        out_shape=[
            jax.ShapeDtypeStruct((N * NS, WD), jnp.float32),
            jax.ShapeDtypeStruct((8, WD), jnp.float32),
        ],
    )(g3, qkv, stats1, g1, beta1, w1, b1)


def _passd_body(w2_ref, g_ref, st_ref, g2_ref, be2_ref, w2w_ref, b2_ref,
                o_ref):
    st = st_ref[...]
    mu = st[0:1, :] / M
    var = st[1:2, :] / M - mu * mu
    alpha = g2_ref[...] * lax.rsqrt(var + EPS)
    shift = be2_ref[...] - mu * alpha
    u = jnp.maximum(w2_ref[...] * alpha + shift, 0.0)
    w3 = lax.dot_general(u, w2w_ref[...], (((1,), (0,)), ((), ())),
                         preferred_element_type=jnp.float32) + b2_ref[...]
    logits = w3.reshape(_BR, NS, WD)
    mx = jnp.max(logits, axis=1, keepdims=True)
    e = jnp.exp(logits - mx)
    sm = e / jnp.sum(e, axis=1, keepdims=True)
    wt = jnp.concatenate([sm] * (D // WD), axis=2)
    o_ref[...] = jnp.sum(g_ref[...] * wt, axis=1)


def _passd(w2, g3, stats2, g2, beta2, w2w, b2):
    return pl.pallas_call(
        _passd_body,
        grid=(N // _BR,),
        in_specs=[
            pl.BlockSpec((_BR * NS, WD), lambda i: (i, 0)),
            pl.BlockSpec((_BR, NS, D), lambda i: (i, 0, 1)),
            pl.BlockSpec((8, WD), lambda i: (0, 0)),
            pl.BlockSpec((1, WD), lambda i: (0, 0)),
            pl.BlockSpec((1, WD), lambda i: (0, 0)),
            pl.BlockSpec((WD, WD), lambda i: (0, 0)),
            pl.BlockSpec((1, WD), lambda i: (0, 0)),
        ],
        out_specs=pl.BlockSpec((_BR, D), lambda i: (i, 0)),
        out_shape=jax.ShapeDtypeStruct((N, D), jnp.float32),
    )(w2, g3, stats2, g2, beta2, w2w, b2)


def kernel(p, x, o, Wq, bq, Wk, bk, Wv, bv, Wp1, bp1, gP, betaP, Wp2, bp2,
           g1, beta1, W1, b1, g2, beta2, W2, b2):
    pp = jnp.concatenate(
        [p, jnp.full((NP - N, 3), PADC, jnp.float32)], axis=0)
    xp = jnp.concatenate(
        [x, jnp.zeros((NP - N, D), jnp.float32)], axis=0)
    ppt = pp.T
    wqkv = jnp.concatenate([Wq, Wk, Wv], axis=1)
    bqkv = jnp.concatenate([bq, bk, bv])[None, :]

    qkv = _qkv(xp, wqkv, bqkv)
    idx = _knn(pp, ppt)

    idxf = idx[:N].reshape(-1)
    npad = 32 * 5120 - N * NS
    idxp = jnp.concatenate(
        [idxf, jnp.full((npad,), NP - 1, jnp.int32)])

    ones_h = jnp.ones((128,), jnp.float32)
    zeros_h = jnp.zeros((NP,), jnp.float32)
    counts = _sc_counts(idxp, ones_h, zeros_h)
    counts_col = counts[:, None]

    sp = _statsp(pp, counts_col, Wp1, bp1[None, :])
    ab = _a2(pp, qkv, sp, gP[None, :], betaP[None, :], Wp1, bp1[None, :],
             Wp2, bp2[None, :])

    gfull = _sc_gather(ab, idxp)
    g3 = gfull[: N * NS].reshape(N, NS, 2 * D)

    st1 = _statsb(g3, qkv)
    w2, st2 = _passc(g3, qkv, st1, g1[None, :], beta1[None, :], W1,
                     b1[None, :])
    out = _passd(w2, g3, st2, g2[None, :], beta2[None, :], W2, b2[None, :])
    return out


# speedup vs baseline: 4.3455x; 1.0269x over previous
import functools

import jax
import jax.numpy as jnp
from jax import lax
from jax.experimental import pallas as pl
from jax.experimental.pallas import tpu as pltpu
from jax.experimental.pallas import tpu_sc as plsc

N = 10000
NP = 10240
NS = 16
D = 128
WD = 16
EPS = 1e-5
M = float(N * NS)
PADC = 1e4
BIG = 3e38


def _qkv_body(x_ref, w_ref, b_ref, o_ref):
    o_ref[...] = (
        jnp.dot(x_ref[...], w_ref[...], preferred_element_type=jnp.float32)
        + b_ref[...]
    )


def _qkv(xp, wqkv, bqkv):
    R = 512
    return pl.pallas_call(
        _qkv_body,
        grid=(NP // R,),
        in_specs=[
            pl.BlockSpec((R, D), lambda i: (i, 0)),
            pl.BlockSpec((D, 3 * D), lambda i: (0, 0)),
            pl.BlockSpec((1, 3 * D), lambda i: (0, 0)),
        ],
        out_specs=pl.BlockSpec((R, 3 * D), lambda i: (i, 0)),
        out_shape=jax.ShapeDtypeStruct((NP, 3 * D), jnp.float32),
    )(xp, wqkv, bqkv)


_KR = 256


def _knn_body(pb_ref, pt_ref, o_ref, scr):
    pb = pb_ref[...]
    pt = pt_ref[...]
    sqr = jnp.sum(pb * pb, axis=1, keepdims=True)
    sqc = jnp.sum(pt * pt, axis=0, keepdims=True)
    mm = lax.dot_general(pb, pt, (((1,), (0,)), ((), ())),
                         preferred_element_type=jnp.float32)
    scr[...] = sqr + sqc - 2.0 * mm
    cols = lax.broadcasted_iota(jnp.int32, (_KR, NP), 1).astype(jnp.float32)
    outs = []
    for _ in range(NS):
        d = scr[...]
        m = jnp.min(d, axis=1, keepdims=True)
        j = jnp.min(jnp.where(d == m, cols, BIG), axis=1, keepdims=True)
        outs.append(j)
        scr[...] = jnp.where(cols == j, BIG, d)
    o_ref[...] = jnp.concatenate(outs, axis=1).astype(jnp.int32)


def _knn(pp, ppt):
    return pl.pallas_call(
        _knn_body,
        grid=(NP // _KR,),
        in_specs=[
            pl.BlockSpec((_KR, 3), lambda i: (i, 0)),
            pl.BlockSpec((3, NP), lambda i: (0, 0)),
        ],
        out_specs=pl.BlockSpec((_KR, NS), lambda i: (i, 0)),
        out_shape=jax.ShapeDtypeStruct((NP, NS), jnp.int32),
        scratch_shapes=[pltpu.VMEM((_KR, NP), jnp.float32)],
    )(pp, ppt)


def _sc_counts(idxp, ones_h, zeros_h):
    total = idxp.shape[0]
    tpc = 16
    bpt = total // tpc
    ch = 128
    nch = bpt // ch
    mesh = plsc.VectorSubcoreMesh(core_axis_name="c", subcore_axis_name="s")

    @functools.partial(
        pl.kernel,
        mesh=mesh,
        out_type=jax.ShapeDtypeStruct((NP,), jnp.float32),
        scratch_types=[
            pltpu.VMEM((ch,), jnp.int32),
            pltpu.VMEM((ch,), jnp.float32),
            pltpu.VMEM_SHARED((NP,), jnp.float32),
        ],
    )
    def k(idx_hbm, ones_hbm, zeros_hbm, out_hbm, idx_v, ones_v, shared):
        cid = lax.axis_index("c")
        sid = lax.axis_index("s")

        @pl.when(cid == 0)
        def _():
            @pl.when(sid == 0)
            def _():
                pltpu.sync_copy(zeros_hbm, shared)

            plsc.subcore_barrier()
            pltpu.sync_copy(ones_hbm, ones_v)
            base = sid * bpt

            def chunk(c, carry):
                off = base + c * ch
                pltpu.sync_copy(idx_hbm.at[pl.ds(off, ch)], idx_v)
                pltpu.sync_copy(ones_v, shared.at[idx_v], add=True)
                return carry

            lax.fori_loop(0, nch, chunk, 0)
            plsc.subcore_barrier()

            @pl.when(sid == 0)
            def _():
                pltpu.sync_copy(shared, out_hbm)

    return k(idxp, ones_h, zeros_h)


def _sc_gather(ab, idxp):
    total = idxp.shape[0]
    nw = 32
    bpw = total // nw
    ch = 128
    nch = bpw // ch
    mesh = plsc.VectorSubcoreMesh(core_axis_name="c", subcore_axis_name="s")

    @functools.partial(
        pl.kernel,
        mesh=mesh,
        out_type=jax.ShapeDtypeStruct((total, 2 * D), jnp.float32),
        scratch_types=[
            pltpu.VMEM((bpw,), jnp.int32),
            pltpu.VMEM((ch, 2 * D), jnp.float32),
            pltpu.VMEM((ch, 2 * D), jnp.float32),
            pltpu.SemaphoreType.DMA,
            pltpu.SemaphoreType.DMA,
        ],
    )
    def k(ab_hbm, idx_hbm, out_hbm, idx_all, rows_a, rows_b, sem_a, sem_b):
        wid = lax.axis_index("s") * 2 + lax.axis_index("c")
        base = wid * bpw
        pltpu.sync_copy(idx_hbm.at[pl.ds(base, bpw)], idx_all)

        def gcopy(c, buf, sem):
            return pltpu.make_async_copy(
                ab_hbm.at[idx_all.at[pl.ds(c * ch, ch)]], buf, sem)

        gcopy(0, rows_a, sem_a).start()

        def pair(i, carry):
            c = 2 * i

            @pl.when(c + 1 < nch)
            def _():
                gcopy(c + 1, rows_b, sem_b).start()

            gcopy(c, rows_a, sem_a).wait()
            pltpu.sync_copy(rows_a, out_hbm.at[pl.ds(base + c * ch, ch)])

            @pl.when(c + 2 < nch)
            def _():
                gcopy(c + 2, rows_a, sem_a).start()

            @pl.when(c + 1 < nch)
            def _():
                gcopy(c + 1, rows_b, sem_b).wait()
                pltpu.sync_copy(
                    rows_b, out_hbm.at[pl.ds(base + (c + 1) * ch, ch)])

            return carry

        lax.fori_loop(0, (nch + 1) // 2, pair, 0)

    return k(ab, idxp)


_SR = 1024


def _statsp_body(p_ref, c_ref, wp1_ref, bp1_ref, o_ref):
    i = pl.program_id(0)
    pb = p_ref[...]
    c = c_ref[...]
    rows = (lax.broadcasted_iota(jnp.int32, (_SR, 1), 0).astype(jnp.float32)
            + i * float(_SR))
    cm = jnp.where(rows < float(N), c, 0.0)
    h = lax.dot_general(pb, wp1_ref[...], (((1,), (0,)), ((), ())),
                        preferred_element_type=jnp.float32) + bp1_ref[...]
    s0 = jnp.sum(cm * h, axis=0, keepdims=True)
    s1 = jnp.sum(cm * h * h, axis=0, keepdims=True)
    z = jnp.zeros((1, D - 3), jnp.float32)
    part = jnp.concatenate(
        [jnp.concatenate([s0, z], axis=1),
         jnp.concatenate([s1, z], axis=1),
         jnp.zeros((6, D), jnp.float32)],
        axis=0,
    )

    @pl.when(i == 0)
    def _():
        o_ref[...] = jnp.zeros_like(o_ref)

    o_ref[...] += part


def _statsp(pp, counts_col, wp1, bp1):
    return pl.pallas_call(
        _statsp_body,
        grid=(NP // _SR,),
        in_specs=[
            pl.BlockSpec((_SR, 3), lambda i: (i, 0)),
            pl.BlockSpec((_SR, 1), lambda i: (i, 0)),
            pl.BlockSpec((3, 3), lambda i: (0, 0)),
            pl.BlockSpec((1, 3), lambda i: (0, 0)),
        ],
        out_specs=pl.BlockSpec((8, D), lambda i: (0, 0)),
        out_shape=jax.ShapeDtypeStruct((8, D), jnp.float32),
    )(pp, counts_col, wp1, bp1)


_AR = 512


def _a2_body(p_ref, k_ref, v_ref, sp_ref, gp_ref, bP_ref, wp1_ref, bp1_ref,
             wp2_ref, bp2_ref, o_ref):
    s = sp_ref[...]
    mu = s[0:1, 0:3] / M
    var = s[1:2, 0:3] / M - mu * mu
    alpha = gp_ref[...] * lax.rsqrt(var + EPS)
    shift = bP_ref[...] - mu * alpha
    pb = p_ref[...]
    h = lax.dot_general(pb, wp1_ref[...], (((1,), (0,)), ((), ())),
                        preferred_element_type=jnp.float32) + bp1_ref[...]
    r = jnp.maximum(h * alpha + shift, 0.0)
    pr = lax.dot_general(r, wp2_ref[...], (((1,), (0,)), ((), ())),
                         preferred_element_type=jnp.float32) + bp2_ref[...]
    o_ref[...] = jnp.concatenate([k_ref[...] + pr, v_ref[...] + pr], axis=1)


def _a2(pp, qkv, statsp, gp, betap, wp1, bp1, wp2, bp2):
    return pl.pallas_call(
        _a2_body,
        grid=(NP // _AR,),
        in_specs=[
            pl.BlockSpec((_AR, 3), lambda i: (i, 0)),
            pl.BlockSpec((_AR, D), lambda i: (i, 1)),
            pl.BlockSpec((_AR, D), lambda i: (i, 2)),
            pl.BlockSpec((8, D), lambda i: (0, 0)),
            pl.BlockSpec((1, 3), lambda i: (0, 0)),
            pl.BlockSpec((1, 3), lambda i: (0, 0)),
            pl.BlockSpec((3, 3), lambda i: (0, 0)),
            pl.BlockSpec((1, 3), lambda i: (0, 0)),
            pl.BlockSpec((3, D), lambda i: (0, 0)),
            pl.BlockSpec((1, D), lambda i: (0, 0)),
        ],
        out_specs=pl.BlockSpec((_AR, 2 * D), lambda i: (i, 0)),
        out_shape=jax.ShapeDtypeStruct((NP, 2 * D), jnp.float32),
    )(pp, qkv, qkv, statsp, gp, betap, wp1, bp1, wp2, bp2)


_BR = 200


def _statsb_body(g_ref, q_ref, o_ref):
    w = g_ref[...] - q_ref[...][:, None, :]
    s0 = jnp.sum(w, axis=(0, 1))[None, :]
    s1 = jnp.sum(w * w, axis=(0, 1))[None, :]
    part = jnp.concatenate([s0, s1, jnp.zeros((6, D), jnp.float32)], axis=0)

    @pl.when(pl.program_id(0) == 0)
    def _():
        o_ref[...] = jnp.zeros_like(o_ref)

    o_ref[...] += part


def _statsb(g3, qkv):
    return pl.pallas_call(
        _statsb_body,
        grid=(N // _BR,),
        in_specs=[
            pl.BlockSpec((_BR, NS, D), lambda i: (i, 0, 0)),
            pl.BlockSpec((_BR, D), lambda i: (i, 0)),
        ],
        out_specs=pl.BlockSpec((8, D), lambda i: (0, 0)),
        out_shape=jax.ShapeDtypeStruct((8, D), jnp.float32),
    )(g3, qkv)


def _passc_body(g_ref, q_ref, st_ref, g1_ref, be1_ref, w1_ref, b1_ref,
                o_ref, so_ref):
    st = st_ref[...]
    mu = st[0:1, :] / M
    var = st[1:2, :] / M - mu * mu
    alpha = g1_ref[...] * lax.rsqrt(var + EPS)
    shift = be1_ref[...] - mu * alpha
    w = g_ref[...] - q_ref[...][:, None, :]
    w = jnp.maximum(w * alpha[None] + shift[None], 0.0)
    w2 = lax.dot_general(w.reshape(_BR * NS, D), w1_ref[...],
                         (((1,), (0,)), ((), ())),
                         preferred_element_type=jnp.float32) + b1_ref[...]
    o_ref[...] = w2
    s0 = jnp.sum(w2, axis=0, keepdims=True)
    s1 = jnp.sum(w2 * w2, axis=0, keepdims=True)
    part = jnp.concatenate([s0, s1, jnp.zeros((6, WD), jnp.float32)], axis=0)

    @pl.when(pl.program_id(0) == 0)
    def _():
        so_ref[...] = jnp.zeros_like(so_ref)

    so_ref[...] += part


def _passc(g3, qkv, stats1, g1, beta1, w1, b1):
    return pl.pallas_call(
        _passc_body,
        grid=(N // _BR,),
        in_specs=[
            pl.BlockSpec((_BR, NS, D), lambda i: (i, 0, 0)),
            pl.BlockSpec((_BR, D), lambda i: (i, 0)),
            pl.BlockSpec((8, D), lambda i: (0, 0)),
            pl.BlockSpec((1, D), lambda i: (0, 0)),
            pl.BlockSpec((1, D), lambda i: (0, 0)),
            pl.BlockSpec((D, WD), lambda i: (0, 0)),
            pl.BlockSpec((1, WD), lambda i: (0, 0)),
        ],
        out_specs=[
            pl.BlockSpec((_BR * NS, WD), lambda i: (i, 0)),
            pl.BlockSpec((8, WD), lambda i: (0, 0)),
        ],
        out_shape=[
            jax.ShapeDtypeStruct((N * NS, WD), jnp.float32),
            jax.ShapeDtypeStruct((8, WD), jnp.float32),
        ],
    )(g3, qkv, stats1, g1, beta1, w1, b1)


def _passd_body(w2_ref, g_ref, st_ref, g2_ref, be2_ref, w2w_ref, b2_ref,
                o_ref):
    st = st_ref[...]
    mu = st[0:1, :] / M
    var = st[1:2, :] / M - mu * mu
    alpha = g2_ref[...] * lax.rsqrt(var + EPS)
    shift = be2_ref[...] - mu * alpha
    u = jnp.maximum(w2_ref[...] * alpha + shift, 0.0)
    w3 = lax.dot_general(u, w2w_ref[...], (((1,), (0,)), ((), ())),
                         preferred_element_type=jnp.float32) + b2_ref[...]
    logits = w3.reshape(_BR, NS, WD)
    mx = jnp.max(logits, axis=1, keepdims=True)
    e = jnp.exp(logits - mx)
    sm = e / jnp.sum(e, axis=1, keepdims=True)
    wt = jnp.concatenate([sm] * (D // WD), axis=2)
    o_ref[...] = jnp.sum(g_ref[...] * wt, axis=1)


def _passd(w2, g3, stats2, g2, beta2, w2w, b2):
    return pl.pallas_call(
        _passd_body,
        grid=(N // _BR,),
        in_specs=[
            pl.BlockSpec((_BR * NS, WD), lambda i: (i, 0)),
            pl.BlockSpec((_BR, NS, D), lambda i: (i, 0, 1)),
            pl.BlockSpec((8, WD), lambda i: (0, 0)),
            pl.BlockSpec((1, WD), lambda i: (0, 0)),
            pl.BlockSpec((1, WD), lambda i: (0, 0)),
            pl.BlockSpec((WD, WD), lambda i: (0, 0)),
            pl.BlockSpec((1, WD), lambda i: (0, 0)),
        ],
        out_specs=pl.BlockSpec((_BR, D), lambda i: (i, 0)),
        out_shape=jax.ShapeDtypeStruct((N, D), jnp.float32),
    )(w2, g3, stats2, g2, beta2, w2w, b2)


def kernel(p, x, o, Wq, bq, Wk, bk, Wv, bv, Wp1, bp1, gP, betaP, Wp2, bp2,
           g1, beta1, W1, b1, g2, beta2, W2, b2):
    pp = jnp.concatenate(
        [p, jnp.full((NP - N, 3), PADC, jnp.float32)], axis=0)
    xp = jnp.concatenate(
        [x, jnp.zeros((NP - N, D), jnp.float32)], axis=0)
    ppt = pp.T
    wqkv = jnp.concatenate([Wq, Wk, Wv], axis=1)
    bqkv = jnp.concatenate([bq, bk, bv])[None, :]

    qkv = _qkv(xp, wqkv, bqkv)
    idx = _knn(pp, ppt)

    idxf = idx[:N].reshape(-1)
    npad = 32 * 5120 - N * NS
    idxp = jnp.concatenate(
        [idxf, jnp.full((npad,), NP - 1, jnp.int32)])

    ones_h = jnp.ones((128,), jnp.float32)
    zeros_h = jnp.zeros((NP,), jnp.float32)
    counts = _sc_counts(idxp, ones_h, zeros_h)
    counts_col = counts[:, None]

    sp = _statsp(pp, counts_col, Wp1, bp1[None, :])
    ab = _a2(pp, qkv, sp, gP[None, :], betaP[None, :], Wp1, bp1[None, :],
             Wp2, bp2[None, :])

    gfull = _sc_gather(ab, idxp)
    g3 = gfull[: N * NS].reshape(N, NS, 2 * D)

    st1 = _statsb(g3, qkv)
    w2, st2 = _passc(g3, qkv, st1, g1[None, :], beta1[None, :], W1,
                     b1[None, :])
    out = _passd(w2, g3, st2, g2[None, :], beta2[None, :], W2, b2[None, :])
    return out


# speedup vs baseline: 5.0823x; 1.1695x over previous
import functools

import jax
import jax.numpy as jnp
from jax import lax
from jax.experimental import pallas as pl
from jax.experimental.pallas import tpu as pltpu
from jax.experimental.pallas import tpu_sc as plsc

N = 10000
NP = 10240
NS = 16
D = 128
WD = 16
EPS = 1e-5
M = float(N * NS)
PADC = 1e4
BIG = 3e38


def _qkv_body(x_ref, w_ref, b_ref, o_ref):
    o_ref[...] = (
        jnp.dot(x_ref[...], w_ref[...], preferred_element_type=jnp.float32)
        + b_ref[...]
    )


def _qkv(xp, wqkv, bqkv):
    R = 512
    return pl.pallas_call(
        _qkv_body,
        grid=(NP // R,),
        in_specs=[
            pl.BlockSpec((R, D), lambda i: (i, 0)),
            pl.BlockSpec((D, 3 * D), lambda i: (0, 0)),
            pl.BlockSpec((1, 3 * D), lambda i: (0, 0)),
        ],
        out_specs=pl.BlockSpec((R, 3 * D), lambda i: (i, 0)),
        out_shape=jax.ShapeDtypeStruct((NP, 3 * D), jnp.float32),
    )(xp, wqkv, bqkv)


_KR = 128


_CW = 128
_NCH = NP // _CW
_K1 = 5


def _knn_body(pb_ref, pt_ref, o_ref, scr, colscr):
    pb = pb_ref[...]
    pt = pt_ref[...]
    sqr = jnp.sum(pb * pb, axis=1, keepdims=True)
    sqc = jnp.sum(pt * pt, axis=0, keepdims=True)
    mm = lax.dot_general(pb, pt, (((1,), (0,)), ((), ())),
                         preferred_element_type=jnp.float32)
    scr[...] = (sqr + sqc - 2.0 * mm).reshape(_KR, _NCH, _CW)
    colscr[...] = (
        lax.broadcasted_iota(jnp.int32, (_KR, _NCH, _CW), 1) * _CW
        + lax.broadcasted_iota(jnp.int32, (_KR, _NCH, _CW), 2)
    ).astype(jnp.float32)
    vals, idxs = [], []
    for _ in range(_K1):
        d = scr[...]
        cols3 = colscr[...]
        m = jnp.min(d, axis=2, keepdims=True)
        j = jnp.min(jnp.where(d == m, cols3, BIG), axis=2, keepdims=True)
        vals.append(m[:, :, 0])
        idxs.append(j[:, :, 0])
        scr[...] = jnp.where(cols3 == j, BIG, d)
    cv = jnp.concatenate(vals, axis=1)
    ci = jnp.concatenate(idxs, axis=1)
    outs = []
    for _ in range(NS):
        m = jnp.min(cv, axis=1, keepdims=True)
        j = jnp.min(jnp.where(cv == m, ci, BIG), axis=1, keepdims=True)
        outs.append(j)
        cv = jnp.where(ci == j, BIG, cv)
    o_ref[...] = jnp.concatenate(outs, axis=1).astype(jnp.int32)


def _knn(pp, ppt):
    return pl.pallas_call(
        _knn_body,
        grid=(NP // _KR,),
        in_specs=[
            pl.BlockSpec((_KR, 3), lambda i: (i, 0)),
            pl.BlockSpec((3, NP), lambda i: (0, 0)),
        ],
        out_specs=pl.BlockSpec((_KR, NS), lambda i: (i, 0)),
        out_shape=jax.ShapeDtypeStruct((NP, NS), jnp.int32),
        scratch_shapes=[pltpu.VMEM((_KR, _NCH, _CW), jnp.float32),
                        pltpu.VMEM((_KR, _NCH, _CW), jnp.float32)],
    )(pp, ppt)


def _sc_counts(idxp, ones_h, zeros_h):
    total = idxp.shape[0]
    tpc = 16
    bpt = total // tpc
    ch = 128
    nch = bpt // ch
    mesh = plsc.VectorSubcoreMesh(core_axis_name="c", subcore_axis_name="s")

    @functools.partial(
        pl.kernel,
        mesh=mesh,
        out_type=jax.ShapeDtypeStruct((NP,), jnp.float32),
        scratch_types=[
            pltpu.VMEM((ch,), jnp.int32),
            pltpu.VMEM((ch,), jnp.float32),
            pltpu.VMEM_SHARED((NP,), jnp.float32),
        ],
    )
    def k(idx_hbm, ones_hbm, zeros_hbm, out_hbm, idx_v, ones_v, shared):
        cid = lax.axis_index("c")
        sid = lax.axis_index("s")

        @pl.when(cid == 0)
        def _():
            @pl.when(sid == 0)
            def _():
                pltpu.sync_copy(zeros_hbm, shared)

            plsc.subcore_barrier()
            pltpu.sync_copy(ones_hbm, ones_v)
            base = sid * bpt

            def chunk(c, carry):
                off = base + c * ch
                pltpu.sync_copy(idx_hbm.at[pl.ds(off, ch)], idx_v)
                pltpu.sync_copy(ones_v, shared.at[idx_v], add=True)
                return carry

            lax.fori_loop(0, nch, chunk, 0)
            plsc.subcore_barrier()

            @pl.when(sid == 0)
            def _():
                pltpu.sync_copy(shared, out_hbm)

    return k(idxp, ones_h, zeros_h)


def _sc_gather(ab, idxp):
    total = idxp.shape[0]
    nw = 32
    bpw = total // nw
    ch = 128
    nch = bpw // ch
    mesh = plsc.VectorSubcoreMesh(core_axis_name="c", subcore_axis_name="s")

    @functools.partial(
        pl.kernel,
        mesh=mesh,
        out_type=jax.ShapeDtypeStruct((total, 2 * D), jnp.float32),
        scratch_types=[
            pltpu.VMEM((bpw,), jnp.int32),
            pltpu.VMEM((ch, 2 * D), jnp.float32),
            pltpu.VMEM((ch, 2 * D), jnp.float32),
            pltpu.SemaphoreType.DMA,
            pltpu.SemaphoreType.DMA,
        ],
    )
    def k(ab_hbm, idx_hbm, out_hbm, idx_all, rows_a, rows_b, sem_a, sem_b):
        wid = lax.axis_index("s") * 2 + lax.axis_index("c")
        base = wid * bpw
        pltpu.sync_copy(idx_hbm.at[pl.ds(base, bpw)], idx_all)

        def gcopy(c, buf, sem):
            return pltpu.make_async_copy(
                ab_hbm.at[idx_all.at[pl.ds(c * ch, ch)]], buf, sem)

        gcopy(0, rows_a, sem_a).start()

        def pair(i, carry):
            c = 2 * i

            @pl.when(c + 1 < nch)
            def _():
                gcopy(c + 1, rows_b, sem_b).start()

            gcopy(c, rows_a, sem_a).wait()
            pltpu.sync_copy(rows_a, out_hbm.at[pl.ds(base + c * ch, ch)])

            @pl.when(c + 2 < nch)
            def _():
                gcopy(c + 2, rows_a, sem_a).start()

            @pl.when(c + 1 < nch)
            def _():
                gcopy(c + 1, rows_b, sem_b).wait()
                pltpu.sync_copy(
                    rows_b, out_hbm.at[pl.ds(base + (c + 1) * ch, ch)])

            return carry

        lax.fori_loop(0, (nch + 1) // 2, pair, 0)

    return k(ab, idxp)


_SR = 1024


def _statsp_body(p_ref, c_ref, wp1_ref, bp1_ref, o_ref):
    i = pl.program_id(0)
    pb = p_ref[...]
    c = c_ref[...]
    rows = (lax.broadcasted_iota(jnp.int32, (_SR, 1), 0).astype(jnp.float32)
            + i * float(_SR))
    cm = jnp.where(rows < float(N), c, 0.0)
    h = lax.dot_general(pb, wp1_ref[...], (((1,), (0,)), ((), ())),
                        preferred_element_type=jnp.float32) + bp1_ref[...]
    s0 = jnp.sum(cm * h, axis=0, keepdims=True)
    s1 = jnp.sum(cm * h * h, axis=0, keepdims=True)
    z = jnp.zeros((1, D - 3), jnp.float32)
    part = jnp.concatenate(
        [jnp.concatenate([s0, z], axis=1),
         jnp.concatenate([s1, z], axis=1),
         jnp.zeros((6, D), jnp.float32)],
        axis=0,
    )

    @pl.when(i == 0)
    def _():
        o_ref[...] = jnp.zeros_like(o_ref)

    o_ref[...] += part


def _statsp(pp, counts_col, wp1, bp1):
    return pl.pallas_call(
        _statsp_body,
        grid=(NP // _SR,),
        in_specs=[
            pl.BlockSpec((_SR, 3), lambda i: (i, 0)),
            pl.BlockSpec((_SR, 1), lambda i: (i, 0)),
            pl.BlockSpec((3, 3), lambda i: (0, 0)),
            pl.BlockSpec((1, 3), lambda i: (0, 0)),
        ],
        out_specs=pl.BlockSpec((8, D), lambda i: (0, 0)),
        out_shape=jax.ShapeDtypeStruct((8, D), jnp.float32),
    )(pp, counts_col, wp1, bp1)


_AR = 512


def _a2_body(p_ref, k_ref, v_ref, sp_ref, gp_ref, bP_ref, wp1_ref, bp1_ref,
             wp2_ref, bp2_ref, o_ref):
    s = sp_ref[...]
    mu = s[0:1, 0:3] / M
    var = s[1:2, 0:3] / M - mu * mu
    alpha = gp_ref[...] * lax.rsqrt(var + EPS)
    shift = bP_ref[...] - mu * alpha
    pb = p_ref[...]
    h = lax.dot_general(pb, wp1_ref[...], (((1,), (0,)), ((), ())),
                        preferred_element_type=jnp.float32) + bp1_ref[...]
    r = jnp.maximum(h * alpha + shift, 0.0)
    pr = lax.dot_general(r, wp2_ref[...], (((1,), (0,)), ((), ())),
                         preferred_element_type=jnp.float32) + bp2_ref[...]
    o_ref[...] = jnp.concatenate([k_ref[...] + pr, v_ref[...] + pr], axis=1)


def _a2(pp, qkv, statsp, gp, betap, wp1, bp1, wp2, bp2):
    return pl.pallas_call(
        _a2_body,
        grid=(NP // _AR,),
        in_specs=[
            pl.BlockSpec((_AR, 3), lambda i: (i, 0)),
            pl.BlockSpec((_AR, D), lambda i: (i, 1)),
            pl.BlockSpec((_AR, D), lambda i: (i, 2)),
            pl.BlockSpec((8, D), lambda i: (0, 0)),
            pl.BlockSpec((1, 3), lambda i: (0, 0)),
            pl.BlockSpec((1, 3), lambda i: (0, 0)),
            pl.BlockSpec((3, 3), lambda i: (0, 0)),
            pl.BlockSpec((1, 3), lambda i: (0, 0)),
            pl.BlockSpec((3, D), lambda i: (0, 0)),
            pl.BlockSpec((1, D), lambda i: (0, 0)),
        ],
        out_specs=pl.BlockSpec((_AR, 2 * D), lambda i: (i, 0)),
        out_shape=jax.ShapeDtypeStruct((NP, 2 * D), jnp.float32),
    )(pp, qkv, qkv, statsp, gp, betap, wp1, bp1, wp2, bp2)


_BR = 200


def _statsb_body(g_ref, q_ref, o_ref):
    w = g_ref[...] - q_ref[...][:, None, :]
    s0 = jnp.sum(w, axis=(0, 1))[None, :]
    s1 = jnp.sum(w * w, axis=(0, 1))[None, :]
    part = jnp.concatenate([s0, s1, jnp.zeros((6, D), jnp.float32)], axis=0)

    @pl.when(pl.program_id(0) == 0)
    def _():
        o_ref[...] = jnp.zeros_like(o_ref)

    o_ref[...] += part


def _statsb(g3, qkv):
    return pl.pallas_call(
        _statsb_body,
        grid=(N // _BR,),
        in_specs=[
            pl.BlockSpec((_BR, NS, D), lambda i: (i, 0, 0)),
            pl.BlockSpec((_BR, D), lambda i: (i, 0)),
        ],
        out_specs=pl.BlockSpec((8, D), lambda i: (0, 0)),
        out_shape=jax.ShapeDtypeStruct((8, D), jnp.float32),
    )(g3, qkv)


def _passc_body(g_ref, q_ref, st_ref, g1_ref, be1_ref, w1_ref, b1_ref,
                o_ref, so_ref):
    st = st_ref[...]
    mu = st[0:1, :] / M
    var = st[1:2, :] / M - mu * mu
    alpha = g1_ref[...] * lax.rsqrt(var + EPS)
    shift = be1_ref[...] - mu * alpha
    w = g_ref[...] - q_ref[...][:, None, :]
    w = jnp.maximum(w * alpha[None] + shift[None], 0.0)
    w2 = lax.dot_general(w.reshape(_BR * NS, D), w1_ref[...],
                         (((1,), (0,)), ((), ())),
                         preferred_element_type=jnp.float32) + b1_ref[...]
    o_ref[...] = w2
    s0 = jnp.sum(w2, axis=0, keepdims=True)
    s1 = jnp.sum(w2 * w2, axis=0, keepdims=True)
    part = jnp.concatenate([s0, s1, jnp.zeros((6, WD), jnp.float32)], axis=0)

    @pl.when(pl.program_id(0) == 0)
    def _():
        so_ref[...] = jnp.zeros_like(so_ref)

    so_ref[...] += part


def _passc(g3, qkv, stats1, g1, beta1, w1, b1):
    return pl.pallas_call(
        _passc_body,
        grid=(N // _BR,),
        in_specs=[
            pl.BlockSpec((_BR, NS, D), lambda i: (i, 0, 0)),
            pl.BlockSpec((_BR, D), lambda i: (i, 0)),
            pl.BlockSpec((8, D), lambda i: (0, 0)),
            pl.BlockSpec((1, D), lambda i: (0, 0)),
            pl.BlockSpec((1, D), lambda i: (0, 0)),
            pl.BlockSpec((D, WD), lambda i: (0, 0)),
            pl.BlockSpec((1, WD), lambda i: (0, 0)),
        ],
        out_specs=[
            pl.BlockSpec((_BR * NS, WD), lambda i: (i, 0)),
            pl.BlockSpec((8, WD), lambda i: (0, 0)),
        ],
        out_shape=[
            jax.ShapeDtypeStruct((N * NS, WD), jnp.float32),
            jax.ShapeDtypeStruct((8, WD), jnp.float32),
        ],
    )(g3, qkv, stats1, g1, beta1, w1, b1)


def _passd_body(w2_ref, g_ref, st_ref, g2_ref, be2_ref, w2w_ref, b2_ref,
                o_ref):
    st = st_ref[...]
    mu = st[0:1, :] / M
    var = st[1:2, :] / M - mu * mu
    alpha = g2_ref[...] * lax.rsqrt(var + EPS)
    shift = be2_ref[...] - mu * alpha
    u = jnp.maximum(w2_ref[...] * alpha + shift, 0.0)
    w3 = lax.dot_general(u, w2w_ref[...], (((1,), (0,)), ((), ())),
                         preferred_element_type=jnp.float32) + b2_ref[...]
    logits = w3.reshape(_BR, NS, WD)
    mx = jnp.max(logits, axis=1, keepdims=True)
    e = jnp.exp(logits - mx)
    sm = e / jnp.sum(e, axis=1, keepdims=True)
    wt = jnp.concatenate([sm] * (D // WD), axis=2)
    o_ref[...] = jnp.sum(g_ref[...] * wt, axis=1)


def _passd(w2, g3, stats2, g2, beta2, w2w, b2):
    return pl.pallas_call(
        _passd_body,
        grid=(N // _BR,),
        in_specs=[
            pl.BlockSpec((_BR * NS, WD), lambda i: (i, 0)),
            pl.BlockSpec((_BR, NS, D), lambda i: (i, 0, 1)),
            pl.BlockSpec((8, WD), lambda i: (0, 0)),
            pl.BlockSpec((1, WD), lambda i: (0, 0)),
            pl.BlockSpec((1, WD), lambda i: (0, 0)),
            pl.BlockSpec((WD, WD), lambda i: (0, 0)),
            pl.BlockSpec((1, WD), lambda i: (0, 0)),
        ],
        out_specs=pl.BlockSpec((_BR, D), lambda i: (i, 0)),
        out_shape=jax.ShapeDtypeStruct((N, D), jnp.float32),
    )(w2, g3, stats2, g2, beta2, w2w, b2)


def kernel(p, x, o, Wq, bq, Wk, bk, Wv, bv, Wp1, bp1, gP, betaP, Wp2, bp2,
           g1, beta1, W1, b1, g2, beta2, W2, b2):
    pp = jnp.concatenate(
        [p, jnp.full((NP - N, 3), PADC, jnp.float32)], axis=0)
    xp = jnp.concatenate(
        [x, jnp.zeros((NP - N, D), jnp.float32)], axis=0)
    ppt = pp.T
    wqkv = jnp.concatenate([Wq, Wk, Wv], axis=1)
    bqkv = jnp.concatenate([bq, bk, bv])[None, :]

    qkv = _qkv(xp, wqkv, bqkv)
    idx = _knn(pp, ppt)

    idxf = idx[:N].reshape(-1)
    npad = 32 * 5120 - N * NS
    idxp = jnp.concatenate(
        [idxf, jnp.full((npad,), NP - 1, jnp.int32)])

    ones_h = jnp.ones((128,), jnp.float32)
    zeros_h = jnp.zeros((NP,), jnp.float32)
    counts = _sc_counts(idxp, ones_h, zeros_h)
    counts_col = counts[:, None]

    sp = _statsp(pp, counts_col, Wp1, bp1[None, :])
    ab = _a2(pp, qkv, sp, gP[None, :], betaP[None, :], Wp1, bp1[None, :],
             Wp2, bp2[None, :])

    gfull = _sc_gather(ab, idxp)
    g3 = gfull[: N * NS].reshape(N, NS, 2 * D)

    st1 = _statsb(g3, qkv)
    w2, st2 = _passc(g3, qkv, st1, g1[None, :], beta1[None, :], W1,
                     b1[None, :])
    out = _passd(w2, g3, st2, g2[None, :], beta2[None, :], W2, b2[None, :])
    return out


# speedup vs baseline: 5.2771x; 1.0383x over previous
import functools

import jax
import jax.numpy as jnp
from jax import lax
from jax.experimental import pallas as pl
from jax.experimental.pallas import tpu as pltpu
from jax.experimental.pallas import tpu_sc as plsc

N = 10000
NP = 10240
NS = 16
D = 128
WD = 16
KVH = 3 * D
EPS = 1e-5
M = float(N * NS)
PADC = 1e4
BIG = 3e38


def _proj_body(x_ref, p_ref, wq_ref, bq_ref, wkv_ref, bkv_ref, wp1_ref,
               bp1_ref, q_ref, kvh_ref):
    x = x_ref[...]
    q_ref[...] = (
        jnp.dot(x, wq_ref[...], preferred_element_type=jnp.float32)
        + bq_ref[...]
    )
    kv = (jnp.dot(x, wkv_ref[...], preferred_element_type=jnp.float32)
          + bkv_ref[...])
    h = lax.dot_general(p_ref[...], wp1_ref[...], (((1,), (0,)), ((), ())),
                        preferred_element_type=jnp.float32) + bp1_ref[...]
    hp = jnp.concatenate(
        [h, jnp.zeros((h.shape[0], D - 3), jnp.float32)], axis=1)
    kvh_ref[...] = jnp.concatenate([kv, hp], axis=1)


def _proj(xp, pp, wq, bq, wkv, bkv, wp1, bp1):
    R = 512
    return pl.pallas_call(
        _proj_body,
        grid=(NP // R,),
        in_specs=[
            pl.BlockSpec((R, D), lambda i: (i, 0)),
            pl.BlockSpec((R, 3), lambda i: (i, 0)),
            pl.BlockSpec((D, D), lambda i: (0, 0)),
            pl.BlockSpec((1, D), lambda i: (0, 0)),
            pl.BlockSpec((D, 2 * D), lambda i: (0, 0)),
            pl.BlockSpec((1, 2 * D), lambda i: (0, 0)),
            pl.BlockSpec((3, 3), lambda i: (0, 0)),
            pl.BlockSpec((1, 3), lambda i: (0, 0)),
        ],
        out_specs=[
            pl.BlockSpec((R, D), lambda i: (i, 0)),
            pl.BlockSpec((R, KVH), lambda i: (i, 0)),
        ],
        out_shape=[
            jax.ShapeDtypeStruct((NP, D), jnp.float32),
            jax.ShapeDtypeStruct((NP, KVH), jnp.float32),
        ],
    )(xp, pp, wq, bq, wkv, bkv, wp1, bp1)


_KR = 128
_CW = 128
_NCH = NP // _CW
_K1 = 5


def _knn_body(pb_ref, pt_ref, o_ref, scr, colscr):
    pb = pb_ref[...]
    pt = pt_ref[...]
    sqr = jnp.sum(pb * pb, axis=1, keepdims=True)
    sqc = jnp.sum(pt * pt, axis=0, keepdims=True)
    mm = lax.dot_general(pb, pt, (((1,), (0,)), ((), ())),
                         preferred_element_type=jnp.float32)
    scr[...] = (sqr + sqc - 2.0 * mm).reshape(_KR, _NCH, _CW)
    colscr[...] = (
        lax.broadcasted_iota(jnp.int32, (_KR, _NCH, _CW), 1) * _CW
        + lax.broadcasted_iota(jnp.int32, (_KR, _NCH, _CW), 2)
    ).astype(jnp.float32)
    vals, idxs = [], []
    for t in range(_K1):
        d = scr[...]
        cols3 = colscr[...]
        m = jnp.min(d, axis=2, keepdims=True)
        j = jnp.min(jnp.where(d == m, cols3, BIG), axis=2, keepdims=True)
        vals.append(m[:, :, 0])
        idxs.append(j[:, :, 0])
        if t + 1 < _K1:
            scr[...] = jnp.where(cols3 == j, BIG, d)
    cv = jnp.concatenate(vals, axis=1)
    ci = jnp.concatenate(idxs, axis=1)
    outs = []
    for _ in range(NS):
        m = jnp.min(cv, axis=1, keepdims=True)
        j = jnp.min(jnp.where(cv == m, ci, BIG), axis=1, keepdims=True)
        outs.append(j)
        cv = jnp.where(ci == j, BIG, cv)
    o_ref[...] = jnp.concatenate(outs, axis=1).astype(jnp.int32)


def _knn(pp_half, ppt):
    nrows = pp_half.shape[0]
    return pl.pallas_call(
        _knn_body,
        grid=(nrows // _KR,),
        in_specs=[
            pl.BlockSpec((_KR, 3), lambda i: (i, 0)),
            pl.BlockSpec((3, NP), lambda i: (0, 0)),
        ],
        out_specs=pl.BlockSpec((_KR, NS), lambda i: (i, 0)),
        out_shape=jax.ShapeDtypeStruct((nrows, NS), jnp.int32),
        scratch_shapes=[pltpu.VMEM((_KR, _NCH, _CW), jnp.float32),
                        pltpu.VMEM((_KR, _NCH, _CW), jnp.float32)],
    )(pp_half, ppt)


def _sc_gather(kvh, idxp):
    total = idxp.shape[0]
    nw = 32
    bpw = total // nw
    ch = 128
    nch = bpw // ch
    mesh = plsc.VectorSubcoreMesh(core_axis_name="c", subcore_axis_name="s")

    @functools.partial(
        pl.kernel,
        mesh=mesh,
        out_type=jax.ShapeDtypeStruct((total, KVH), jnp.float32),
        scratch_types=[
            pltpu.VMEM((bpw,), jnp.int32),
            pltpu.VMEM((ch, KVH), jnp.float32),
            pltpu.VMEM((ch, KVH), jnp.float32),
            pltpu.SemaphoreType.DMA,
            pltpu.SemaphoreType.DMA,
        ],
    )
    def k(t_hbm, idx_hbm, out_hbm, idx_all, rows_a, rows_b, sem_a, sem_b):
        wid = lax.axis_index("s") * 2 + lax.axis_index("c")
        base = wid * bpw
        pltpu.sync_copy(idx_hbm.at[pl.ds(base, bpw)], idx_all)

        def gcopy(c, buf, sem):
            return pltpu.make_async_copy(
                t_hbm.at[idx_all.at[pl.ds(c * ch, ch)]], buf, sem)

        gcopy(0, rows_a, sem_a).start()

        def pair(i, carry):
            c = 2 * i

            @pl.when(c + 1 < nch)
            def _():
                gcopy(c + 1, rows_b, sem_b).start()

            gcopy(c, rows_a, sem_a).wait()
            pltpu.sync_copy(rows_a, out_hbm.at[pl.ds(base + c * ch, ch)])

            @pl.when(c + 2 < nch)
            def _():
                gcopy(c + 2, rows_a, sem_a).start()

            @pl.when(c + 1 < nch)
            def _():
                gcopy(c + 1, rows_b, sem_b).wait()
                pltpu.sync_copy(
                    rows_b, out_hbm.at[pl.ds(base + (c + 1) * ch, ch)])

            return carry

        lax.fori_loop(0, (nch + 1) // 2, pair, 0)

    return k(kvh, idxp)


_BR = 200


def _statsp_body(g_ref, o_ref):
    h = g_ref[:, :, 0:3]
    s0 = jnp.sum(h, axis=(0, 1))[None, :]
    s1 = jnp.sum(h * h, axis=(0, 1))[None, :]
    z = jnp.zeros((1, 16 - 3), jnp.float32)
    part = jnp.concatenate(
        [jnp.concatenate([s0, z], axis=1),
         jnp.concatenate([s1, z], axis=1),
         jnp.zeros((6, 16), jnp.float32)], axis=0)

    @pl.when(pl.program_id(0) == 0)
    def _():
        o_ref[...] = jnp.zeros_like(o_ref)

    o_ref[...] += part


def _statsp(g3):
    return pl.pallas_call(
        _statsp_body,
        grid=(N // _BR,),
        in_specs=[pl.BlockSpec((_BR, NS, D), lambda i: (i, 0, 2))],
        out_specs=pl.BlockSpec((8, 16), lambda i: (0, 0)),
        out_shape=jax.ShapeDtypeStruct((8, 16), jnp.float32),
    )(g3)


def _pos_term(h_ref, sp, wp2_ref, bp2_ref, gp_ref, bP_ref):
    mu = sp[0:1, 0:3] / M
    var = sp[1:2, 0:3] / M - mu * mu
    alpha = gp_ref[...] * lax.rsqrt(var + EPS)
    shift = bP_ref[...] - mu * alpha
    h = h_ref[:, :, 0:3].reshape(_BR * NS, 3)
    r = jnp.maximum(h * alpha + shift, 0.0)
    return lax.dot_general(r, wp2_ref[...], (((1,), (0,)), ((), ())),
                           preferred_element_type=jnp.float32) + bp2_ref[...]


def _statsb_body(k_ref, h_ref, q_ref, sp_ref, gp_ref, bP_ref, wp2_ref,
                 bp2_ref, o_ref):
    pr = _pos_term(h_ref, sp_ref[...], wp2_ref, bp2_ref, gp_ref, bP_ref)
    w = (k_ref[...] - q_ref[...][:, None, :]).reshape(_BR * NS, D) + pr
    s0 = jnp.sum(w, axis=0, keepdims=True)
    s1 = jnp.sum(w * w, axis=0, keepdims=True)
    part = jnp.concatenate([s0, s1, jnp.zeros((6, D), jnp.float32)], axis=0)

    @pl.when(pl.program_id(0) == 0)
    def _():
        o_ref[...] = jnp.zeros_like(o_ref)

    o_ref[...] += part


def _statsb(g3, q, sp, gp, betap, wp2, bp2):
    return pl.pallas_call(
        _statsb_body,
        grid=(N // _BR,),
        in_specs=[
            pl.BlockSpec((_BR, NS, D), lambda i: (i, 0, 0)),
            pl.BlockSpec((_BR, NS, D), lambda i: (i, 0, 2)),
            pl.BlockSpec((_BR, D), lambda i: (i, 0)),
            pl.BlockSpec((8, 16), lambda i: (0, 0)),
            pl.BlockSpec((1, 3), lambda i: (0, 0)),
            pl.BlockSpec((1, 3), lambda i: (0, 0)),
            pl.BlockSpec((3, D), lambda i: (0, 0)),
            pl.BlockSpec((1, D), lambda i: (0, 0)),
        ],
        out_specs=pl.BlockSpec((8, D), lambda i: (0, 0)),
        out_shape=jax.ShapeDtypeStruct((8, D), jnp.float32),
    )(g3, g3, q, sp, gp, betap, wp2, bp2)


def _passc_body(k_ref, h_ref, q_ref, sp_ref, gp_ref, bP_ref, wp2_ref,
                bp2_ref, st_ref, g1_ref, be1_ref, w1_ref, b1_ref,
                o_ref, so_ref):
    pr = _pos_term(h_ref, sp_ref[...], wp2_ref, bp2_ref, gp_ref, bP_ref)
    st = st_ref[...]
    mu = st[0:1, :] / M
    var = st[1:2, :] / M - mu * mu
    alpha = g1_ref[...] * lax.rsqrt(var + EPS)
    shift = be1_ref[...] - mu * alpha
    w = (k_ref[...] - q_ref[...][:, None, :]).reshape(_BR * NS, D) + pr
    w = jnp.maximum(w * alpha + shift, 0.0)
    w2 = lax.dot_general(w, w1_ref[...], (((1,), (0,)), ((), ())),
                         preferred_element_type=jnp.float32) + b1_ref[...]
    o_ref[...] = w2
    s0 = jnp.sum(w2, axis=0, keepdims=True)
    s1 = jnp.sum(w2 * w2, axis=0, keepdims=True)
    part = jnp.concatenate([s0, s1, jnp.zeros((6, WD), jnp.float32)], axis=0)

    @pl.when(pl.program_id(0) == 0)
    def _():
        so_ref[...] = jnp.zeros_like(so_ref)

    so_ref[...] += part


def _passc(g3, q, sp, gp, betap, wp2, bp2, st1, g1, beta1, w1, b1):
    return pl.pallas_call(
        _passc_body,
        grid=(N // _BR,),
        in_specs=[
            pl.BlockSpec((_BR, NS, D), lambda i: (i, 0, 0)),
            pl.BlockSpec((_BR, NS, D), lambda i: (i, 0, 2)),
            pl.BlockSpec((_BR, D), lambda i: (i, 0)),
            pl.BlockSpec((8, 16), lambda i: (0, 0)),
            pl.BlockSpec((1, 3), lambda i: (0, 0)),
            pl.BlockSpec((1, 3), lambda i: (0, 0)),
            pl.BlockSpec((3, D), lambda i: (0, 0)),
            pl.BlockSpec((1, D), lambda i: (0, 0)),
            pl.BlockSpec((8, D), lambda i: (0, 0)),
            pl.BlockSpec((1, D), lambda i: (0, 0)),
            pl.BlockSpec((1, D), lambda i: (0, 0)),
            pl.BlockSpec((D, WD), lambda i: (0, 0)),
            pl.BlockSpec((1, WD), lambda i: (0, 0)),
        ],
        out_specs=[
            pl.BlockSpec((_BR * NS, WD), lambda i: (i, 0)),
            pl.BlockSpec((8, WD), lambda i: (0, 0)),
        ],
        out_shape=[
            jax.ShapeDtypeStruct((N * NS, WD), jnp.float32),
            jax.ShapeDtypeStruct((8, WD), jnp.float32),
        ],
    )(g3, g3, q, sp, gp, betap, wp2, bp2, st1, g1, beta1, w1, b1)


def _passd_body(w2_ref, v_ref, h_ref, sp_ref, gp_ref, bP_ref, wp2_ref,
                bp2_ref, st_ref, g2_ref, be2_ref, w2w_ref, b2_ref, o_ref):
    st = st_ref[...]
    mu = st[0:1, :] / M
    var = st[1:2, :] / M - mu * mu
    alpha = g2_ref[...] * lax.rsqrt(var + EPS)
    shift = be2_ref[...] - mu * alpha
    u = jnp.maximum(w2_ref[...] * alpha + shift, 0.0)
    w3 = lax.dot_general(u, w2w_ref[...], (((1,), (0,)), ((), ())),
                         preferred_element_type=jnp.float32) + b2_ref[...]
    logits = w3.reshape(_BR, NS, WD)
    mx = jnp.max(logits, axis=1, keepdims=True)
    e = jnp.exp(logits - mx)
    sm = e / jnp.sum(e, axis=1, keepdims=True)
    wt = jnp.concatenate([sm] * (D // WD), axis=2)
    pr = _pos_term(h_ref, sp_ref[...], wp2_ref, bp2_ref, gp_ref, bP_ref)
    v = v_ref[...] + pr.reshape(_BR, NS, D)
    o_ref[...] = jnp.sum(v * wt, axis=1)


def _passd(w2, g3, sp, gp, betap, wp2, bp2, st2, g2, beta2, w2w, b2):
    return pl.pallas_call(
        _passd_body,
        grid=(N // _BR,),
        in_specs=[
            pl.BlockSpec((_BR * NS, WD), lambda i: (i, 0)),
            pl.BlockSpec((_BR, NS, D), lambda i: (i, 0, 1)),
            pl.BlockSpec((_BR, NS, D), lambda i: (i, 0, 2)),
            pl.BlockSpec((8, 16), lambda i: (0, 0)),
            pl.BlockSpec((1, 3), lambda i: (0, 0)),
            pl.BlockSpec((1, 3), lambda i: (0, 0)),
            pl.BlockSpec((3, D), lambda i: (0, 0)),
            pl.BlockSpec((1, D), lambda i: (0, 0)),
            pl.BlockSpec((8, WD), lambda i: (0, 0)),
            pl.BlockSpec((1, WD), lambda i: (0, 0)),
            pl.BlockSpec((1, WD), lambda i: (0, 0)),
            pl.BlockSpec((WD, WD), lambda i: (0, 0)),
            pl.BlockSpec((1, WD), lambda i: (0, 0)),
        ],
        out_specs=pl.BlockSpec((_BR, D), lambda i: (i, 0)),
        out_shape=jax.ShapeDtypeStruct((N, D), jnp.float32),
    )(w2, g3, g3, sp, gp, betap, wp2, bp2, st2, g2, beta2, w2w, b2)


def kernel(p, x, o, Wq, bq, Wk, bk, Wv, bv, Wp1, bp1, gP, betaP, Wp2, bp2,
           g1, beta1, W1, b1, g2, beta2, W2, b2):
    pp = jnp.concatenate(
        [p, jnp.full((NP - N, 3), PADC, jnp.float32)], axis=0)
    xp = jnp.concatenate(
        [x, jnp.zeros((NP - N, D), jnp.float32)], axis=0)
    ppt = pp.T
    wkv = jnp.concatenate([Wk, Wv], axis=1)
    bkv = jnp.concatenate([bk, bv])[None, :]

    q, kvh = _proj(xp, pp, Wq, bq[None, :], wkv, bkv, Wp1, bp1[None, :])

    half = NP // 2
    idx0 = _knn(pp[:half], ppt)
    idx1 = _knn(pp[half:], ppt)
    g0 = _sc_gather(kvh, idx0.reshape(-1))
    g1_ = _sc_gather(kvh, idx1.reshape(-1))

    gfull = jnp.concatenate([g0, g1_], axis=0)[: N * NS]
    g3 = gfull.reshape(N, NS, KVH)

    sp = _statsp(g3)
    gp2 = gP[None, :]
    bp2_ = betaP[None, :]
    st1 = _statsb(g3, q, sp, gp2, bp2_, Wp2, bp2[None, :])
    w2, st2 = _passc(g3, q, sp, gp2, bp2_, Wp2, bp2[None, :], st1,
                     g1[None, :], beta1[None, :], W1, b1[None, :])
    out = _passd(w2, g3, sp, gp2, bp2_, Wp2, bp2[None, :], st2,
                 g2[None, :], beta2[None, :], W2, b2[None, :])
    return out


# speedup vs baseline: 7.2362x; 1.3712x over previous
import functools

import jax
import jax.numpy as jnp
from jax import lax
from jax.experimental import pallas as pl
from jax.experimental.pallas import tpu as pltpu
from jax.experimental.pallas import tpu_sc as plsc

N = 10000
NP = 10240
NS = 16
D = 128
WD = 16
KVH = 3 * D
EPS = 1e-5
M = float(N * NS)
PADC = 1e4
BIG = 3e38


def _proj_body(x_ref, p_ref, wq_ref, bq_ref, wkv_ref, bkv_ref, wp1_ref,
               bp1_ref, q_ref, kvh_ref):
    x = x_ref[...]
    q_ref[...] = (
        jnp.dot(x, wq_ref[...], preferred_element_type=jnp.float32)
        + bq_ref[...]
    )
    kv = (jnp.dot(x, wkv_ref[...], preferred_element_type=jnp.float32)
          + bkv_ref[...])
    h = lax.dot_general(p_ref[...], wp1_ref[...], (((1,), (0,)), ((), ())),
                        preferred_element_type=jnp.float32) + bp1_ref[...]
    hp = jnp.concatenate(
        [h, jnp.zeros((h.shape[0], D - 3), jnp.float32)], axis=1)
    kvh_ref[...] = jnp.concatenate([kv, hp], axis=1)


def _proj(xp, pp, wq, bq, wkv, bkv, wp1, bp1):
    R = 512
    return pl.pallas_call(
        _proj_body,
        grid=(NP // R,),
        in_specs=[
            pl.BlockSpec((R, D), lambda i: (i, 0)),
            pl.BlockSpec((R, 3), lambda i: (i, 0)),
            pl.BlockSpec((D, D), lambda i: (0, 0)),
            pl.BlockSpec((1, D), lambda i: (0, 0)),
            pl.BlockSpec((D, 2 * D), lambda i: (0, 0)),
            pl.BlockSpec((1, 2 * D), lambda i: (0, 0)),
            pl.BlockSpec((3, 3), lambda i: (0, 0)),
            pl.BlockSpec((1, 3), lambda i: (0, 0)),
        ],
        out_specs=[
            pl.BlockSpec((R, D), lambda i: (i, 0)),
            pl.BlockSpec((R, KVH), lambda i: (i, 0)),
        ],
        out_shape=[
            jax.ShapeDtypeStruct((NP, D), jnp.float32),
            jax.ShapeDtypeStruct((NP, KVH), jnp.float32),
        ],
    )(xp, pp, wq, bq, wkv, bkv, wp1, bp1)


_KR = 128
_CW = 128
_NCH = NP // _CW
_K1 = 5


def _knn_body(pb_ref, pt_ref, o_ref, scr, colscr):
    pb = pb_ref[...]
    pt = pt_ref[...]
    sqr = jnp.sum(pb * pb, axis=1, keepdims=True)
    sqc = jnp.sum(pt * pt, axis=0, keepdims=True)
    mm = lax.dot_general(pb, pt, (((1,), (0,)), ((), ())),
                         preferred_element_type=jnp.float32)
    scr[...] = (sqr + sqc - 2.0 * mm).reshape(_KR, _NCH, _CW)
    colscr[...] = (
        lax.broadcasted_iota(jnp.int32, (_KR, _NCH, _CW), 1) * _CW
        + lax.broadcasted_iota(jnp.int32, (_KR, _NCH, _CW), 2)
    ).astype(jnp.float32)
    vals, idxs = [], []
    for t in range(_K1):
        d = scr[...]
        cols3 = colscr[...]
        m = jnp.min(d, axis=1, keepdims=True)
        j = jnp.min(jnp.where(d == m, cols3, BIG), axis=1, keepdims=True)
        vals.append(m[:, 0, :])
        idxs.append(j[:, 0, :])
        if t + 1 < _K1:
            scr[...] = jnp.where(cols3 == j, BIG, d)
    cv = jnp.concatenate(vals, axis=1)
    ci = jnp.concatenate(idxs, axis=1)
    outs = []
    for _ in range(NS):
        m = jnp.min(cv, axis=1, keepdims=True)
        j = jnp.min(jnp.where(cv == m, ci, BIG), axis=1, keepdims=True)
        outs.append(j)
        cv = jnp.where(ci == j, BIG, cv)
    o_ref[...] = jnp.concatenate(outs, axis=1).astype(jnp.int32)


def _knn(pp_half, ppt):
    nrows = pp_half.shape[0]
    return pl.pallas_call(
        _knn_body,
        grid=(nrows // _KR,),
        in_specs=[
            pl.BlockSpec((_KR, 3), lambda i: (i, 0)),
            pl.BlockSpec((3, NP), lambda i: (0, 0)),
        ],
        out_specs=pl.BlockSpec((_KR, NS), lambda i: (i, 0)),
        out_shape=jax.ShapeDtypeStruct((nrows, NS), jnp.int32),
        scratch_shapes=[pltpu.VMEM((_KR, _NCH, _CW), jnp.float32),
                        pltpu.VMEM((_KR, _NCH, _CW), jnp.float32)],
    )(pp_half, ppt)


def _sc_gather(kvh, idxp):
    total = idxp.shape[0]
    nw = 32
    bpw = total // nw
    ch = 128
    nch = bpw // ch
    mesh = plsc.VectorSubcoreMesh(core_axis_name="c", subcore_axis_name="s")

    @functools.partial(
        pl.kernel,
        mesh=mesh,
        out_type=jax.ShapeDtypeStruct((total, KVH), jnp.float32),
        scratch_types=[
            pltpu.VMEM((bpw,), jnp.int32),
            pltpu.VMEM((ch, KVH), jnp.float32),
            pltpu.VMEM((ch, KVH), jnp.float32),
            pltpu.SemaphoreType.DMA,
            pltpu.SemaphoreType.DMA,
        ],
    )
    def k(t_hbm, idx_hbm, out_hbm, idx_all, rows_a, rows_b, sem_a, sem_b):
        wid = lax.axis_index("s") * 2 + lax.axis_index("c")
        base = wid * bpw
        pltpu.sync_copy(idx_hbm.at[pl.ds(base, bpw)], idx_all)

        def gcopy(c, buf, sem):
            return pltpu.make_async_copy(
                t_hbm.at[idx_all.at[pl.ds(c * ch, ch)]], buf, sem)

        gcopy(0, rows_a, sem_a).start()

        def pair(i, carry):
            c = 2 * i

            @pl.when(c + 1 < nch)
            def _():
                gcopy(c + 1, rows_b, sem_b).start()

            gcopy(c, rows_a, sem_a).wait()
            pltpu.sync_copy(rows_a, out_hbm.at[pl.ds(base + c * ch, ch)])

            @pl.when(c + 2 < nch)
            def _():
                gcopy(c + 2, rows_a, sem_a).start()

            @pl.when(c + 1 < nch)
            def _():
                gcopy(c + 1, rows_b, sem_b).wait()
                pltpu.sync_copy(
                    rows_b, out_hbm.at[pl.ds(base + (c + 1) * ch, ch)])

            return carry

        lax.fori_loop(0, (nch + 1) // 2, pair, 0)

    return k(kvh, idxp)


_BR = 200


def _statsp_body(g_ref, o_ref):
    h = g_ref[:, :, 0:3]
    s0 = jnp.sum(h, axis=(0, 1))[None, :]
    s1 = jnp.sum(h * h, axis=(0, 1))[None, :]
    z = jnp.zeros((1, 16 - 3), jnp.float32)
    part = jnp.concatenate(
        [jnp.concatenate([s0, z], axis=1),
         jnp.concatenate([s1, z], axis=1),
         jnp.zeros((6, 16), jnp.float32)], axis=0)

    @pl.when(pl.program_id(0) == 0)
    def _():
        o_ref[...] = jnp.zeros_like(o_ref)

    o_ref[...] += part


def _statsp(g3):
    return pl.pallas_call(
        _statsp_body,
        grid=(N // _BR,),
        in_specs=[pl.BlockSpec((_BR, NS, D), lambda i: (i, 0, 2))],
        out_specs=pl.BlockSpec((8, 16), lambda i: (0, 0)),
        out_shape=jax.ShapeDtypeStruct((8, 16), jnp.float32),
    )(g3)


def _pos_term(h_ref, sp, wp2_ref, bp2_ref, gp_ref, bP_ref):
    mu = sp[0:1, 0:3] / M
    var = sp[1:2, 0:3] / M - mu * mu
    alpha = gp_ref[...] * lax.rsqrt(var + EPS)
    shift = bP_ref[...] - mu * alpha
    h = h_ref[:, :, 0:3].reshape(_BR * NS, 3)
    r = jnp.maximum(h * alpha + shift, 0.0)
    return lax.dot_general(r, wp2_ref[...], (((1,), (0,)), ((), ())),
                           preferred_element_type=jnp.float32) + bp2_ref[...]


def _statsb_body(k_ref, h_ref, q_ref, sp_ref, gp_ref, bP_ref, wp2_ref,
                 bp2_ref, o_ref):
    pr = _pos_term(h_ref, sp_ref[...], wp2_ref, bp2_ref, gp_ref, bP_ref)
    w = (k_ref[...] - q_ref[...][:, None, :]).reshape(_BR * NS, D) + pr
    s0 = jnp.sum(w, axis=0, keepdims=True)
    s1 = jnp.sum(w * w, axis=0, keepdims=True)
    part = jnp.concatenate([s0, s1, jnp.zeros((6, D), jnp.float32)], axis=0)

    @pl.when(pl.program_id(0) == 0)
    def _():
        o_ref[...] = jnp.zeros_like(o_ref)

    o_ref[...] += part


def _statsb(g3, q, sp, gp, betap, wp2, bp2):
    return pl.pallas_call(
        _statsb_body,
        grid=(N // _BR,),
        in_specs=[
            pl.BlockSpec((_BR, NS, D), lambda i: (i, 0, 0)),
            pl.BlockSpec((_BR, NS, D), lambda i: (i, 0, 2)),
            pl.BlockSpec((_BR, D), lambda i: (i, 0)),
            pl.BlockSpec((8, 16), lambda i: (0, 0)),
            pl.BlockSpec((1, 3), lambda i: (0, 0)),
            pl.BlockSpec((1, 3), lambda i: (0, 0)),
            pl.BlockSpec((3, D), lambda i: (0, 0)),
            pl.BlockSpec((1, D), lambda i: (0, 0)),
        ],
        out_specs=pl.BlockSpec((8, D), lambda i: (0, 0)),
        out_shape=jax.ShapeDtypeStruct((8, D), jnp.float32),
    )(g3, g3, q, sp, gp, betap, wp2, bp2)


def _passc_body(k_ref, h_ref, q_ref, sp_ref, gp_ref, bP_ref, wp2_ref,
                bp2_ref, st_ref, g1_ref, be1_ref, w1_ref, b1_ref,
                o_ref, so_ref):
    pr = _pos_term(h_ref, sp_ref[...], wp2_ref, bp2_ref, gp_ref, bP_ref)
    st = st_ref[...]
    mu = st[0:1, :] / M
    var = st[1:2, :] / M - mu * mu
    alpha = g1_ref[...] * lax.rsqrt(var + EPS)
    shift = be1_ref[...] - mu * alpha
    w = (k_ref[...] - q_ref[...][:, None, :]).reshape(_BR * NS, D) + pr
    w = jnp.maximum(w * alpha + shift, 0.0)
    w2 = lax.dot_general(w, w1_ref[...], (((1,), (0,)), ((), ())),
                         preferred_element_type=jnp.float32) + b1_ref[...]
    o_ref[...] = w2
    s0 = jnp.sum(w2, axis=0, keepdims=True)
    s1 = jnp.sum(w2 * w2, axis=0, keepdims=True)
    part = jnp.concatenate([s0, s1, jnp.zeros((6, WD), jnp.float32)], axis=0)

    @pl.when(pl.program_id(0) == 0)
    def _():
        so_ref[...] = jnp.zeros_like(so_ref)

    so_ref[...] += part


def _passc(g3, q, sp, gp, betap, wp2, bp2, st1, g1, beta1, w1, b1):
    return pl.pallas_call(
        _passc_body,
        grid=(N // _BR,),
        in_specs=[
            pl.BlockSpec((_BR, NS, D), lambda i: (i, 0, 0)),
            pl.BlockSpec((_BR, NS, D), lambda i: (i, 0, 2)),
            pl.BlockSpec((_BR, D), lambda i: (i, 0)),
            pl.BlockSpec((8, 16), lambda i: (0, 0)),
            pl.BlockSpec((1, 3), lambda i: (0, 0)),
            pl.BlockSpec((1, 3), lambda i: (0, 0)),
            pl.BlockSpec((3, D), lambda i: (0, 0)),
            pl.BlockSpec((1, D), lambda i: (0, 0)),
            pl.BlockSpec((8, D), lambda i: (0, 0)),
            pl.BlockSpec((1, D), lambda i: (0, 0)),
            pl.BlockSpec((1, D), lambda i: (0, 0)),
            pl.BlockSpec((D, WD), lambda i: (0, 0)),
            pl.BlockSpec((1, WD), lambda i: (0, 0)),
        ],
        out_specs=[
            pl.BlockSpec((_BR * NS, WD), lambda i: (i, 0)),
            pl.BlockSpec((8, WD), lambda i: (0, 0)),
        ],
        out_shape=[
            jax.ShapeDtypeStruct((N * NS, WD), jnp.float32),
            jax.ShapeDtypeStruct((8, WD), jnp.float32),
        ],
    )(g3, g3, q, sp, gp, betap, wp2, bp2, st1, g1, beta1, w1, b1)


def _passd_body(w2_ref, v_ref, h_ref, sp_ref, gp_ref, bP_ref, wp2_ref,
                bp2_ref, st_ref, g2_ref, be2_ref, w2w_ref, b2_ref, o_ref):
    st = st_ref[...]
    mu = st[0:1, :] / M
    var = st[1:2, :] / M - mu * mu
    alpha = g2_ref[...] * lax.rsqrt(var + EPS)
    shift = be2_ref[...] - mu * alpha
    u = jnp.maximum(w2_ref[...] * alpha + shift, 0.0)
    w3 = lax.dot_general(u, w2w_ref[...], (((1,), (0,)), ((), ())),
                         preferred_element_type=jnp.float32) + b2_ref[...]
    logits = w3.reshape(_BR, NS, WD)
    mx = jnp.max(logits, axis=1, keepdims=True)
    e = jnp.exp(logits - mx)
    sm = e / jnp.sum(e, axis=1, keepdims=True)
    wt = jnp.concatenate([sm] * (D // WD), axis=2)
    pr = _pos_term(h_ref, sp_ref[...], wp2_ref, bp2_ref, gp_ref, bP_ref)
    v = v_ref[...] + pr.reshape(_BR, NS, D)
    o_ref[...] = jnp.sum(v * wt, axis=1)


def _passd(w2, g3, sp, gp, betap, wp2, bp2, st2, g2, beta2, w2w, b2):
    return pl.pallas_call(
        _passd_body,
        grid=(N // _BR,),
        in_specs=[
            pl.BlockSpec((_BR * NS, WD), lambda i: (i, 0)),
            pl.BlockSpec((_BR, NS, D), lambda i: (i, 0, 1)),
            pl.BlockSpec((_BR, NS, D), lambda i: (i, 0, 2)),
            pl.BlockSpec((8, 16), lambda i: (0, 0)),
            pl.BlockSpec((1, 3), lambda i: (0, 0)),
            pl.BlockSpec((1, 3), lambda i: (0, 0)),
            pl.BlockSpec((3, D), lambda i: (0, 0)),
            pl.BlockSpec((1, D), lambda i: (0, 0)),
            pl.BlockSpec((8, WD), lambda i: (0, 0)),
            pl.BlockSpec((1, WD), lambda i: (0, 0)),
            pl.BlockSpec((1, WD), lambda i: (0, 0)),
            pl.BlockSpec((WD, WD), lambda i: (0, 0)),
            pl.BlockSpec((1, WD), lambda i: (0, 0)),
        ],
        out_specs=pl.BlockSpec((_BR, D), lambda i: (i, 0)),
        out_shape=jax.ShapeDtypeStruct((N, D), jnp.float32),
    )(w2, g3, g3, sp, gp, betap, wp2, bp2, st2, g2, beta2, w2w, b2)


def kernel(p, x, o, Wq, bq, Wk, bk, Wv, bv, Wp1, bp1, gP, betaP, Wp2, bp2,
           g1, beta1, W1, b1, g2, beta2, W2, b2):
    pp = jnp.concatenate(
        [p, jnp.full((NP - N, 3), PADC, jnp.float32)], axis=0)
    xp = jnp.concatenate(
        [x, jnp.zeros((NP - N, D), jnp.float32)], axis=0)
    ppt = pp.T
    wkv = jnp.concatenate([Wk, Wv], axis=1)
    bkv = jnp.concatenate([bk, bv])[None, :]

    q, kvh = _proj(xp, pp, Wq, bq[None, :], wkv, bkv, Wp1, bp1[None, :])

    half = NP // 2
    idx0 = _knn(pp[:half], ppt)
    idx1 = _knn(pp[half:], ppt)
    g0 = _sc_gather(kvh, idx0.reshape(-1))
    g1_ = _sc_gather(kvh, idx1.reshape(-1))

    gfull = jnp.concatenate([g0, g1_], axis=0)[: N * NS]
    g3 = gfull.reshape(N, NS, KVH)

    sp = _statsp(g3)
    gp2 = gP[None, :]
    bp2_ = betaP[None, :]
    st1 = _statsb(g3, q, sp, gp2, bp2_, Wp2, bp2[None, :])
    w2, st2 = _passc(g3, q, sp, gp2, bp2_, Wp2, bp2[None, :], st1,
                     g1[None, :], beta1[None, :], W1, b1[None, :])
    out = _passd(w2, g3, sp, gp2, bp2_, Wp2, bp2[None, :], st2,
                 g2[None, :], beta2[None, :], W2, b2[None, :])
    return out
